# baseline (speedup 1.0000x reference)
import jax
import jax.numpy as jnp
from jax.experimental import pallas as pl
from jax.experimental.pallas import tpu as pltpu

K = 8192
D = 128
BETA = 0.25
TEMP = 0.1
DECAY = 0.9
LOG2E = 1.4426950408889634
BLOCK = 512


def _vq_kernel(z_ref, e_ref, ema_ref, zq_ref, stats_ref,
               enorm_ref, psum_ref, cnt_ref, acc_ref):
    i = pl.program_id(0)
    nsteps = pl.num_programs(0)
    n_rows = nsteps * BLOCK

    @pl.when(i == 0)
    def _init():
        e = e_ref[...]
        nrm = jnp.sqrt(jnp.sum(e * e, axis=1, keepdims=True))
        enorm_ref[...] = e / jnp.maximum(nrm, 1e-12)
        psum_ref[...] = jnp.zeros_like(psum_ref)
        cnt_ref[...] = jnp.zeros_like(cnt_ref)
        acc_ref[0] = 0.0

    z = z_ref[...]
    znrm = jnp.sqrt(jnp.sum(z * z, axis=1, keepdims=True))
    zn = z / jnp.maximum(znrm, 1e-12)
    cos = jax.lax.dot_general(zn, enorm_ref[...], (((1,), (1,)), ((), ())),
                              preferred_element_type=jnp.float32)
    col = jax.lax.broadcasted_iota(jnp.int32, (BLOCK, K), 1)
    idx = jnp.argmax(cos, axis=1)[:, None]
    onehot = (col == idx).astype(jnp.bfloat16)
    el = jnp.exp2(cos * (LOG2E / TEMP))
    onecf = jnp.ones((K, 1), jnp.float32)
    s = jax.lax.dot_general(el, onecf, (((1,), (0,)), ((), ())),
                            preferred_element_type=jnp.float32)
    sinv_row = jnp.transpose(1.0 / s, (1, 0))
    psum_ref[...] += jax.lax.dot_general(sinv_row, el, (((1,), (0,)), ((), ())),
                                         preferred_element_type=jnp.float32)
    oner = jnp.ones((1, BLOCK), jnp.bfloat16)
    cnt_ref[...] += jax.lax.dot_general(oner, onehot, (((1,), (0,)), ((), ())),
                                        preferred_element_type=jnp.float32)
    zq = jax.lax.dot_general(onehot, e_ref[...], (((1,), (0,)), ((), ())),
                             preferred_element_type=jnp.float32)
    dlt = zq - z
    zq_ref[...] = z + dlt
    acc_ref[0] += jnp.sum(dlt * dlt)

    @pl.when(i == nsteps - 1)
    def _fin():
        e_mean = cnt_ref[...] * (1.0 / n_rows)
        perplexity = jnp.exp(-jnp.sum(e_mean * jnp.log(e_mean + 1e-8)))
        p_avg = psum_ref[...] * (1.0 / n_rows) + 1e-8
        entropy = -jnp.sum(p_avg * jnp.log(p_avg))
        commit = acc_ref[0] * ((1.0 + BETA) / (n_rows * D))
        new_ema = DECAY * ema_ref[...] + (1.0 - DECAY) * e_mean
        new_ema = jnp.where(new_ema < (0.0125 / K), 1.0 / K, new_ema)
        ema_min = jnp.min(new_ema)
        lane = jax.lax.broadcasted_iota(jnp.int32, (1, 128), 1)
        stats_ref[...] = jnp.where(
            lane == 0, commit,
            jnp.where(lane == 1, perplexity,
                      jnp.where(lane == 2, entropy,
                                jnp.where(lane == 3, ema_min, 0.0))))


def kernel(z, embedding_weight, codebook_probs_ema):
    orig_shape = z.shape
    n = z.size // D
    z_flat = z.reshape(n, D)
    ema2d = codebook_probs_ema.reshape(1, K)
    grid = n // BLOCK
    zq, stats = pl.pallas_call(
        _vq_kernel,
        grid=(grid,),
        in_specs=[
            pl.BlockSpec((BLOCK, D), lambda i: (i, 0)),
            pl.BlockSpec((K, D), lambda i: (0, 0)),
            pl.BlockSpec((1, K), lambda i: (0, 0)),
        ],
        out_specs=[
            pl.BlockSpec((BLOCK, D), lambda i: (i, 0)),
            pl.BlockSpec((1, 128), lambda i: (0, 0)),
        ],
        out_shape=[
            jax.ShapeDtypeStruct((n, D), jnp.float32),
            jax.ShapeDtypeStruct((1, 128), jnp.float32),
        ],
        scratch_shapes=[
            pltpu.VMEM((K, D), jnp.float32),
            pltpu.VMEM((1, K), jnp.float32),
            pltpu.VMEM((1, K), jnp.float32),
            pltpu.SMEM((1,), jnp.float32),
        ],
    )(z_flat, embedding_weight, ema2d)
    return (zq.reshape(orig_shape), stats[0, 0], stats[0, 1],
            stats[0, 2], stats[0, 3])


# speedup vs baseline: 1.0187x; 1.0187x over previous
import jax
import jax.numpy as jnp
from jax.experimental import pallas as pl
from jax.experimental.pallas import tpu as pltpu

K = 8192
D = 128
BETA = 0.25
TEMP = 0.1
DECAY = 0.9
LOG2E = 1.4426950408889634
BLOCK = 512


def _vq_kernel(z_ref, e_ref, ema_ref, zq_ref, stats_ref,
               enorm_ref, psum_ref, cnt_ref, acc_ref):
    i = pl.program_id(0)
    nsteps = pl.num_programs(0)
    n_rows = nsteps * BLOCK

    @pl.when(i == 0)
    def _init():
        e = e_ref[...]
        nrm = jnp.sqrt(jnp.sum(e * e, axis=1, keepdims=True))
        enorm_ref[...] = e / jnp.maximum(nrm, 1e-12)
        psum_ref[...] = jnp.zeros_like(psum_ref)
        cnt_ref[...] = jnp.zeros_like(cnt_ref)
        acc_ref[0] = 0.0

    z = z_ref[...]
    znrm = jnp.sqrt(jnp.sum(z * z, axis=1, keepdims=True))
    zn = z / jnp.maximum(znrm, 1e-12)
    cos = jax.lax.dot_general(zn, enorm_ref[...], (((1,), (1,)), ((), ())),
                              preferred_element_type=jnp.float32)
    col = jax.lax.broadcasted_iota(jnp.int32, (BLOCK, K), 1)
    idx = jnp.argmax(cos, axis=1)[:, None]
    onehot = (col == idx).astype(jnp.float32)
    el = jnp.exp2(cos * (LOG2E / TEMP))
    onecf = jnp.ones((K, 1), jnp.float32)
    s = jax.lax.dot_general(el, onecf, (((1,), (0,)), ((), ())),
                            preferred_element_type=jnp.float32)
    sinv_row = jnp.transpose(1.0 / s, (1, 0))
    psum_ref[...] += jax.lax.dot_general(sinv_row, el, (((1,), (0,)), ((), ())),
                                         preferred_element_type=jnp.float32)
    oner = jnp.ones((1, BLOCK), jnp.float32)
    cnt_ref[...] += jax.lax.dot_general(oner, onehot, (((1,), (0,)), ((), ())),
                                        preferred_element_type=jnp.float32)
    zq = jax.lax.dot_general(onehot, e_ref[...], (((1,), (0,)), ((), ())),
                             preferred_element_type=jnp.float32)
    dlt = zq - z
    zq_ref[...] = z + dlt
    acc_ref[0] += jnp.sum(dlt * dlt)

    @pl.when(i == nsteps - 1)
    def _fin():
        e_mean = cnt_ref[...] * (1.0 / n_rows)
        perplexity = jnp.exp(-jnp.sum(e_mean * jnp.log(e_mean + 1e-8)))
        p_avg = psum_ref[...] * (1.0 / n_rows) + 1e-8
        entropy = -jnp.sum(p_avg * jnp.log(p_avg))
        commit = acc_ref[0] * ((1.0 + BETA) / (n_rows * D))
        new_ema = DECAY * ema_ref[...] + (1.0 - DECAY) * e_mean
        new_ema = jnp.where(new_ema < (0.0125 / K), 1.0 / K, new_ema)
        ema_min = jnp.min(new_ema)
        lane = jax.lax.broadcasted_iota(jnp.int32, (1, 128), 1)
        stats_ref[...] = jnp.where(
            lane == 0, commit,
            jnp.where(lane == 1, perplexity,
                      jnp.where(lane == 2, entropy,
                                jnp.where(lane == 3, ema_min, 0.0))))


def kernel(z, embedding_weight, codebook_probs_ema):
    orig_shape = z.shape
    n = z.size // D
    z_flat = z.reshape(n, D)
    ema2d = codebook_probs_ema.reshape(1, K)
    grid = n // BLOCK
    zq, stats = pl.pallas_call(
        _vq_kernel,
        grid=(grid,),
        in_specs=[
            pl.BlockSpec((BLOCK, D), lambda i: (i, 0)),
            pl.BlockSpec((K, D), lambda i: (0, 0)),
            pl.BlockSpec((1, K), lambda i: (0, 0)),
        ],
        out_specs=[
            pl.BlockSpec((BLOCK, D), lambda i: (i, 0)),
            pl.BlockSpec((1, 128), lambda i: (0, 0)),
        ],
        out_shape=[
            jax.ShapeDtypeStruct((n, D), jnp.float32),
            jax.ShapeDtypeStruct((1, 128), jnp.float32),
        ],
        scratch_shapes=[
            pltpu.VMEM((K, D), jnp.float32),
            pltpu.VMEM((1, K), jnp.float32),
            pltpu.VMEM((1, K), jnp.float32),
            pltpu.SMEM((1,), jnp.float32),
        ],
    )(z_flat, embedding_weight, ema2d)
    return (zq.reshape(orig_shape), stats[0, 0], stats[0, 1],
            stats[0, 2], stats[0, 3])
